# baseline (speedup 1.0000x reference)
import functools

import jax
import jax.numpy as jnp
from jax import lax
from jax.experimental import pallas as pl
from jax.experimental.pallas import tpu as pltpu
from jax.experimental.pallas import tpu_sc as plsc

N_NODE = 10000
D = 128
E = 320000
INV_T = float(1.0 / (128.0 ** 0.5))

NC = 2
NS = 16
NW = NC * NS
CHUNK = 80
EB = E // NW
NB_B = EB // CHUNK
EC = E // NS
NB_C = EC // CHUNK


def _proj_body(x_ref, wk_ref, wv_ref, k_ref, v_ref):
    x = x_ref[...]
    dn = (((1,), (1,)), ((), ()))
    k_ref[...] = lax.dot_general(x, wk_ref[...], dn,
                                 preferred_element_type=jnp.float32)
    v_ref[...] = lax.dot_general(x, wv_ref[...], dn,
                                 preferred_element_type=jnp.float32)


def _project(x, wk, wv):
    nb = 10
    rb = N_NODE // nb
    return pl.pallas_call(
        _proj_body,
        grid=(nb,),
        in_specs=[
            pl.BlockSpec((rb, D), lambda i: (i, 0)),
            pl.BlockSpec((D, D), lambda i: (0, 0)),
            pl.BlockSpec((D, D), lambda i: (0, 0)),
        ],
        out_specs=[
            pl.BlockSpec((rb, D), lambda i: (i, 0)),
            pl.BlockSpec((rb, D), lambda i: (i, 0)),
        ],
        out_shape=[
            jax.ShapeDtypeStruct((N_NODE, D), jnp.float32),
            jax.ShapeDtypeStruct((N_NODE, D), jnp.float32),
        ],
    )(x, wk, wv)


def _logits_body(a_hbm, b_hbm, segl_hbm, segr_hbm, ex_hbm,
                 idxl_v, idxr_v, ex_v, arow0, brow0, arow1, brow1,
                 arow2, brow2, arow3, brow3, sbuf_v,
                 sem0, sem1, sem2, sem3):
    cid = lax.axis_index("c")
    sid = lax.axis_index("s")
    wid = sid * NC + cid
    pltpu.sync_copy(segl_hbm.at[wid], idxl_v)
    pltpu.sync_copy(segr_hbm.at[wid], idxr_v)
    lanes = lax.iota(jnp.int32, 16)

    def fire(c, ar, br, sem):
        pltpu.async_copy(a_hbm.at[idxl_v.at[c]], ar, sem)
        pltpu.async_copy(b_hbm.at[idxr_v.at[c]], br, sem)

    def drain(c, ar, br, sem):
        pltpu.make_async_copy(a_hbm.at[idxl_v.at[c]], ar, sem).wait()
        pltpu.make_async_copy(b_hbm.at[idxr_v.at[c]], br, sem).wait()

    def compute(c, ar, br):
        def grp_body(g, carry2):
            r0 = g * 16
            for e in range(16):
                row = r0 + e
                p = ar[row, pl.ds(0, 16)] * br[row, pl.ds(0, 16)]
                for k in range(1, 8):
                    p = p + (ar[row, pl.ds(k * 16, 16)]
                             * br[row, pl.ds(k * 16, 16)])
                sbuf_v[e, pl.ds(0, 16)] = p
            tv = plsc.load_gather(
                sbuf_v, [lanes, jnp.full((16,), 0, jnp.int32)])
            for l in range(1, 16):
                tv = tv + plsc.load_gather(
                    sbuf_v, [lanes, jnp.full((16,), l, jnp.int32)])
            ex_v[pl.ds(c * CHUNK + r0, 16)] = jnp.exp(tv * INV_T)
            return carry2

        lax.fori_loop(0, CHUNK // 16, grp_body, None)

    bufs = [(arow0, brow0, sem0), (arow1, brow1, sem1),
            (arow2, brow2, sem2), (arow3, brow3, sem3)]
    fire(0, arow0, brow0, sem0)
    fire(1, arow1, brow1, sem1)
    fire(2, arow2, brow2, sem2)

    def quad_body(i, carry):
        c0 = 4 * i
        for q in range(4):
            ar, br, sem = bufs[q]
            c = c0 + q
            drain(c, ar, br, sem)
            compute(c, ar, br)
            fa, fb, fsem = bufs[(q + 3) % 4]

            @pl.when(c + 3 < NB_B)
            def _():
                fire(c + 3, fa, fb, fsem)

        return carry

    lax.fori_loop(0, NB_B // 4, quad_body, None)
    ar, br, sem = bufs[(NB_B - 1) % 4]
    drain(NB_B - 1, ar, br, sem)
    compute(NB_B - 1, ar, br)
    pltpu.sync_copy(ex_v, ex_hbm.at[pl.ds(wid * EB, EB)])


def _edge_logits(a, b, segl3, segr3):
    mesh = plsc.VectorSubcoreMesh(core_axis_name="c", subcore_axis_name="s")
    f = functools.partial(
        pl.kernel,
        mesh=mesh,
        compiler_params=pltpu.CompilerParams(needs_layout_passes=False),
        out_type=jax.ShapeDtypeStruct((E,), jnp.float32),
        scratch_types=[
            pltpu.VMEM((NB_B, CHUNK), jnp.int32),
            pltpu.VMEM((NB_B, CHUNK), jnp.int32),
            pltpu.VMEM((EB,), jnp.float32),
            pltpu.VMEM((CHUNK, D), jnp.float32),
            pltpu.VMEM((CHUNK, D), jnp.float32),
            pltpu.VMEM((CHUNK, D), jnp.float32),
            pltpu.VMEM((CHUNK, D), jnp.float32),
            pltpu.VMEM((CHUNK, D), jnp.float32),
            pltpu.VMEM((CHUNK, D), jnp.float32),
            pltpu.VMEM((CHUNK, D), jnp.float32),
            pltpu.VMEM((CHUNK, D), jnp.float32),
            pltpu.VMEM((16, 16), jnp.float32),
            pltpu.SemaphoreType.DMA,
            pltpu.SemaphoreType.DMA,
            pltpu.SemaphoreType.DMA,
            pltpu.SemaphoreType.DMA,
        ],
    )(_logits_body)
    return f(a, b, segl3, segr3)


CC = 80
NBC = EC // CC
NZB = N_NODE // CC
SCN = 5
NSC = NBC // SCN
DZB = 640
NDP = NS * DZB


def _scatter_body(vr_hbm, vl_hbm, seglc_hbm, segrc_hbm, exc_hbm,
                  outv_hbm, outd_hbm,
                  gidx0, sidx0, exv0, gidx1, sidx1, exv1,
                  vrow0, vrow1, srow0, srow1, zden_v,
                  acc_sh, den_sh,
                  sem0, sem1, sems0, sems1, semd0, semd1, stg0, stg1):
    cid = lax.axis_index("c")
    sid = lax.axis_index("s")

    def zrow_body(i, carry):
        for k in range(D // 16):
            vrow0[i, pl.ds(k * 16, 16)] = jnp.zeros((16,), jnp.float32)
        return carry

    lax.fori_loop(0, CC, zrow_body, None)

    def zden_body(i, carry):
        zden_v[pl.ds(i * 16, 16)] = jnp.zeros((16,), jnp.float32)
        return carry

    lax.fori_loop(0, DZB // 16, zden_body, None)

    for k in range(8):
        b = sid + NS * k

        @pl.when(b < NZB)
        def _():
            pltpu.sync_copy(vrow0, acc_sh.at[pl.ds(b * CC, CC)])

    pltpu.sync_copy(zden_v, den_sh.at[pl.ds(sid * DZB, DZB)])

    plsc.subcore_barrier()

    def run_side(v_hbm, g4d, s4d, out_slot):
        def stage(s, gx, sx, ex, stg):
            pltpu.async_copy(g4d.at[sid, s], gx, stg)
            pltpu.async_copy(s4d.at[sid, s], sx, stg)
            pltpu.async_copy(exc_hbm.at[sid, s], ex, stg)

        def stage_drain(s, gx, sx, ex, stg):
            pltpu.make_async_copy(g4d.at[sid, s], gx, stg).wait()
            pltpu.make_async_copy(s4d.at[sid, s], sx, stg).wait()
            pltpu.make_async_copy(exc_hbm.at[sid, s], ex, stg).wait()

        def process(s, gidx_v, sidx_v, ex_v, stg):
            stage_drain(s, gidx_v, sidx_v, ex_v, stg)

            def fire(j, buf, sem):
                pltpu.async_copy(v_hbm.at[gidx_v.at[j]], buf, sem)

            def drain(j, buf, sem):
                pltpu.make_async_copy(
                    v_hbm.at[gidx_v.at[j]], buf, sem).wait()

            def drain_scatter(j, sbuf, sems, semd):
                pltpu.make_async_copy(
                    sbuf, acc_sh.at[sidx_v.at[j]], sems).wait()
                pltpu.make_async_copy(
                    ex_v.at[j], den_sh.at[sidx_v.at[j]], semd).wait()

            def scale_scatter(j, buf, sbuf, sems, semd):
                jvec = jnp.full((16,), 0, jnp.int32) + j

                def do_edge(row):
                    exb = plsc.load_gather(
                        ex_v, [jvec, jnp.full((16,), 0, jnp.int32) + row])
                    for k in range(8):
                        sbuf[row, pl.ds(k * 16, 16)] = (
                            buf[row, pl.ds(k * 16, 16)] * exb)

                def grp_body(g, carry3):
                    for e in range(16):
                        do_edge(g * 16 + e)
                    return carry3

                lax.fori_loop(0, CC // 16, grp_body, None)
                for e in range(CC - 16 * (CC // 16)):
                    do_edge(16 * (CC // 16) + e)
                pltpu.async_copy(sbuf, acc_sh.at[sidx_v.at[j]], sems,
                                 add=True)
                pltpu.async_copy(ex_v.at[j], den_sh.at[sidx_v.at[j]],
                                 semd, add=True)

            vbufs = [(vrow0, sem0), (vrow1, sem1)]
            sbufs = [(srow0, sems0, semd0), (srow1, sems1, semd1)]
            fire(0, vrow0, sem0)
            fire(1, vrow1, sem1)
            for j in range(SCN):
                vb, vsem = vbufs[j % 2]
                sb, ssem, dsem = sbufs[j % 2]
                drain(j, vb, vsem)
                if j >= 2:
                    drain_scatter(j - 2, sb, ssem, dsem)
                scale_scatter(j, vb, sb, ssem, dsem)
                if j + 2 < SCN:
                    fire(j + 2, vb, vsem)
            sb, ssem, dsem = sbufs[(SCN - 2) % 2]
            drain_scatter(SCN - 2, sb, ssem, dsem)
            sb, ssem, dsem = sbufs[(SCN - 1) % 2]
            drain_scatter(SCN - 1, sb, ssem, dsem)

        stage(0, gidx0, sidx0, exv0, stg0)

        def super_pair(i, carry):
            s0 = 2 * i
            stage(s0 + 1, gidx1, sidx1, exv1, stg1)
            process(s0, gidx0, sidx0, exv0, stg0)

            @pl.when(s0 + 2 < NSC)
            def _():
                stage(s0 + 2, gidx0, sidx0, exv0, stg0)

            process(s0 + 1, gidx1, sidx1, exv1, stg1)
            return carry

        lax.fori_loop(0, NSC // 2, super_pair, None)
        plsc.subcore_barrier()
        for k in range(8):
            b = sid + NS * k

            @pl.when(b < NZB)
            def _():
                pltpu.sync_copy(acc_sh.at[pl.ds(b * CHUNK, CHUNK)],
                                outv_hbm.at[out_slot, pl.ds(b * CHUNK, CHUNK)])

        pltpu.sync_copy(
            den_sh.at[pl.ds(sid * DZB, DZB)],
            outd_hbm.at[pl.ds(out_slot * NDP + sid * DZB, DZB)])

    @pl.when(cid == 0)
    def _():
        run_side(vr_hbm, segrc_hbm, seglc_hbm, 0)

    @pl.when(cid == 1)
    def _():
        run_side(vl_hbm, seglc_hbm, segrc_hbm, 1)


def _edge_scatter(vr, vl, seglc, segrc, exc):
    mesh = plsc.VectorSubcoreMesh(core_axis_name="c", subcore_axis_name="s")
    f = functools.partial(
        pl.kernel,
        mesh=mesh,
        compiler_params=pltpu.CompilerParams(needs_layout_passes=False),
        out_type=[
            jax.ShapeDtypeStruct((2, N_NODE, D), jnp.float32),
            jax.ShapeDtypeStruct((2 * NDP,), jnp.float32),
        ],
        scratch_types=[
            pltpu.VMEM((SCN, CC), jnp.int32),
            pltpu.VMEM((SCN, CC), jnp.int32),
            pltpu.VMEM((SCN, CC), jnp.float32),
            pltpu.VMEM((SCN, CC), jnp.int32),
            pltpu.VMEM((SCN, CC), jnp.int32),
            pltpu.VMEM((SCN, CC), jnp.float32),
            pltpu.VMEM((CC, D), jnp.float32),
            pltpu.VMEM((CC, D), jnp.float32),
            pltpu.VMEM((CC, D), jnp.float32),
            pltpu.VMEM((CC, D), jnp.float32),
            pltpu.VMEM((DZB,), jnp.float32),
            pltpu.VMEM_SHARED((N_NODE, D), jnp.float32),
            pltpu.VMEM_SHARED((NDP,), jnp.float32),
            pltpu.SemaphoreType.DMA,
            pltpu.SemaphoreType.DMA,
            pltpu.SemaphoreType.DMA,
            pltpu.SemaphoreType.DMA,
            pltpu.SemaphoreType.DMA,
            pltpu.SemaphoreType.DMA,
            pltpu.SemaphoreType.DMA,
            pltpu.SemaphoreType.DMA,
        ],
    )(_scatter_body)
    return f(vr, vl, seglc, segrc, exc)


def _out_body(acc_ref, den_ref, wo_ref, bo_ref, o_ref):
    av = acc_ref[0]
    d = den_ref[0]
    m = lax.dot_general(av, wo_ref[...], (((1,), (1,)), ((), ())),
                        preferred_element_type=jnp.float32)
    safe = jnp.where(d > 0, d, 1.0)
    r = m / safe + bo_ref[...]
    o_ref[0] = jnp.where(r >= 0, r, 0.01 * r)


def _finalize(acc, den, wo, bo2):
    nb = 10
    rb = N_NODE // nb
    return pl.pallas_call(
        _out_body,
        grid=(2, nb),
        in_specs=[
            pl.BlockSpec((1, rb, D), lambda s, i: (s, i, 0)),
            pl.BlockSpec((1, rb, 1), lambda s, i: (s, i, 0)),
            pl.BlockSpec((D, D), lambda s, i: (0, 0)),
            pl.BlockSpec((1, D), lambda s, i: (0, 0)),
        ],
        out_specs=pl.BlockSpec((1, rb, D), lambda s, i: (s, i, 0)),
        out_shape=jax.ShapeDtypeStruct((2, N_NODE, D), jnp.float32),
    )(acc, den, wo, bo2)


def kernel(node_left, segmentation_index_left, index_left, node_right,
           segmentation_index_right, index_right, Wk, Wv, Wo, bo):
    seg_l = segmentation_index_left
    seg_r = segmentation_index_right

    a, vl = _project(node_left, Wk, Wv)
    b, vr = _project(node_right, Wk, Wv)

    segl_b = seg_l.reshape(NW, NB_B, CHUNK)
    segr_b = seg_r.reshape(NW, NB_B, CHUNK)
    ex = _edge_logits(a, b, segl_b, segr_b)

    segl_c = seg_l.reshape(NS, NSC, SCN, CC)
    segr_c = seg_r.reshape(NS, NSC, SCN, CC)
    exc = ex.reshape(NS, NSC, SCN, CC)
    accv, den = _edge_scatter(vr, vl, segl_c, segr_c, exc)
    den = den.reshape(2, NDP)[:, :N_NODE]

    out = _finalize(accv, den.reshape(2, N_NODE, 1), Wo, bo.reshape(1, D))
    return (out[0], out[1])


# speedup vs baseline: 1.0330x; 1.0330x over previous
import functools

import jax
import jax.numpy as jnp
from jax import lax
from jax.experimental import pallas as pl
from jax.experimental.pallas import tpu as pltpu
from jax.experimental.pallas import tpu_sc as plsc

N_NODE = 10000
D = 128
E = 320000
INV_T = float(1.0 / (128.0 ** 0.5))

NC = 2
NS = 16
NW = NC * NS
CHUNK = 80
EB = E // NW
NB_B = EB // CHUNK
EC = E // NS
NB_C = EC // CHUNK


def _proj_body(xl_ref, xr_ref, wk_ref, wv_ref, a_ref, vl_ref, b_ref, vr_ref):
    xl = xl_ref[...]
    xr = xr_ref[...]
    wk = wk_ref[...]
    wv = wv_ref[...]
    dn = (((1,), (1,)), ((), ()))
    a_ref[...] = lax.dot_general(xl, wk, dn,
                                 preferred_element_type=jnp.float32)
    vl_ref[...] = lax.dot_general(xl, wv, dn,
                                  preferred_element_type=jnp.float32)
    b_ref[...] = lax.dot_general(xr, wk, dn,
                                 preferred_element_type=jnp.float32)
    vr_ref[...] = lax.dot_general(xr, wv, dn,
                                  preferred_element_type=jnp.float32)


def _project(xl, xr, wk, wv):
    nb = 10
    rb = N_NODE // nb
    blk = pl.BlockSpec((rb, D), lambda i: (i, 0))
    wblk = pl.BlockSpec((D, D), lambda i: (0, 0))
    osh = jax.ShapeDtypeStruct((N_NODE, D), jnp.float32)
    return pl.pallas_call(
        _proj_body,
        grid=(nb,),
        in_specs=[blk, blk, wblk, wblk],
        out_specs=[blk, blk, blk, blk],
        out_shape=[osh, osh, osh, osh],
    )(xl, xr, wk, wv)


def _logits_body(a_hbm, b_hbm, segl_hbm, segr_hbm, ex_hbm,
                 idxl_v, idxr_v, ex_v, arow0, brow0, arow1, brow1, sbuf_v,
                 sem0, sem1):
    cid = lax.axis_index("c")
    sid = lax.axis_index("s")
    wid = sid * NC + cid
    pltpu.sync_copy(segl_hbm.at[pl.ds(wid * EB, EB)], idxl_v)
    pltpu.sync_copy(segr_hbm.at[pl.ds(wid * EB, EB)], idxr_v)
    lanes = lax.iota(jnp.int32, 16)

    def fire(c, ar, br, sem):
        pltpu.async_copy(
            a_hbm.at[idxl_v.at[pl.ds(c * CHUNK, CHUNK)]], ar, sem)
        pltpu.async_copy(
            b_hbm.at[idxr_v.at[pl.ds(c * CHUNK, CHUNK)]], br, sem)

    def drain(c, ar, br, sem):
        pltpu.make_async_copy(
            a_hbm.at[idxl_v.at[pl.ds(c * CHUNK, CHUNK)]], ar, sem).wait()
        pltpu.make_async_copy(
            b_hbm.at[idxr_v.at[pl.ds(c * CHUNK, CHUNK)]], br, sem).wait()

    def compute(c, ar, br):
        def grp_body(g, carry2):
            r0 = g * 16
            for e in range(16):
                row = r0 + e
                p = ar[row, pl.ds(0, 16)] * br[row, pl.ds(0, 16)]
                for k in range(1, 8):
                    p = p + (ar[row, pl.ds(k * 16, 16)]
                             * br[row, pl.ds(k * 16, 16)])
                sbuf_v[e, pl.ds(0, 16)] = p
            tv = plsc.load_gather(
                sbuf_v, [lanes, jnp.full((16,), 0, jnp.int32)])
            for l in range(1, 16):
                tv = tv + plsc.load_gather(
                    sbuf_v, [lanes, jnp.full((16,), l, jnp.int32)])
            ex_v[pl.ds(c * CHUNK + r0, 16)] = jnp.exp(tv * INV_T)
            return carry2

        lax.fori_loop(0, CHUNK // 16, grp_body, None)

    fire(0, arow0, brow0, sem0)

    def pair_body(i, carry):
        c0 = 2 * i
        fire(c0 + 1, arow1, brow1, sem1)
        drain(c0, arow0, brow0, sem0)
        compute(c0, arow0, brow0)
        fire(c0 + 2, arow0, brow0, sem0)
        drain(c0 + 1, arow1, brow1, sem1)
        compute(c0 + 1, arow1, brow1)
        return carry

    lax.fori_loop(0, (NB_B - 1) // 2, pair_body, None)
    drain(NB_B - 1, arow0, brow0, sem0)
    compute(NB_B - 1, arow0, brow0)
    pltpu.sync_copy(ex_v, ex_hbm.at[pl.ds(wid * EB, EB)])


def _edge_logits(a, b, segl3, segr3):
    mesh = plsc.VectorSubcoreMesh(core_axis_name="c", subcore_axis_name="s")
    f = functools.partial(
        pl.kernel,
        mesh=mesh,
        compiler_params=pltpu.CompilerParams(needs_layout_passes=False),
        out_type=jax.ShapeDtypeStruct((E,), jnp.float32),
        scratch_types=[
            pltpu.VMEM((EB,), jnp.int32),
            pltpu.VMEM((EB,), jnp.int32),
            pltpu.VMEM((EB,), jnp.float32),
            pltpu.VMEM((CHUNK, D), jnp.float32),
            pltpu.VMEM((CHUNK, D), jnp.float32),
            pltpu.VMEM((CHUNK, D), jnp.float32),
            pltpu.VMEM((CHUNK, D), jnp.float32),
            pltpu.VMEM((16, 16), jnp.float32),
            pltpu.SemaphoreType.DMA,
            pltpu.SemaphoreType.DMA,
        ],
    )(_logits_body)
    return f(a, b, segl3, segr3)


CC = 80
NBC = EC // CC
NZB = N_NODE // CC
SCN = 5
NSC = NBC // SCN
DZB = 640
NDP = NS * DZB


def _scatter_body(vr_hbm, vl_hbm, seglc_hbm, segrc_hbm, exc_hbm,
                  outv_hbm, outd_hbm,
                  gidx0, sidx0, exv0, gidx1, sidx1, exv1,
                  vrow0, vrow1, srow0, srow1, zden_v,
                  acc_sh, den_sh,
                  sem0, sem1, sems0, sems1, semd0, semd1, stg0, stg1):
    cid = lax.axis_index("c")
    sid = lax.axis_index("s")

    def zrow_body(i, carry):
        for k in range(D // 16):
            vrow0[i, pl.ds(k * 16, 16)] = jnp.zeros((16,), jnp.float32)
        return carry

    lax.fori_loop(0, CC, zrow_body, None)

    def zden_body(i, carry):
        zden_v[pl.ds(i * 16, 16)] = jnp.zeros((16,), jnp.float32)
        return carry

    lax.fori_loop(0, DZB // 16, zden_body, None)

    for k in range(8):
        b = sid + NS * k

        @pl.when(b < NZB)
        def _():
            pltpu.sync_copy(vrow0, acc_sh.at[pl.ds(b * CC, CC)])

    pltpu.sync_copy(zden_v, den_sh.at[pl.ds(sid * DZB, DZB)])

    plsc.subcore_barrier()

    def run_side(v_hbm, g4d, s4d, out_slot):
        def stage(s, gx, sx, ex, stg):
            pltpu.async_copy(g4d.at[sid, s], gx, stg)
            pltpu.async_copy(s4d.at[sid, s], sx, stg)
            pltpu.async_copy(exc_hbm.at[sid, s], ex, stg)

        def stage_drain(s, gx, sx, ex, stg):
            pltpu.make_async_copy(g4d.at[sid, s], gx, stg).wait()
            pltpu.make_async_copy(s4d.at[sid, s], sx, stg).wait()
            pltpu.make_async_copy(exc_hbm.at[sid, s], ex, stg).wait()

        def process(s, gidx_v, sidx_v, ex_v, stg):
            stage_drain(s, gidx_v, sidx_v, ex_v, stg)

            def fire(j, buf, sem):
                pltpu.async_copy(v_hbm.at[gidx_v.at[j]], buf, sem)

            def drain(j, buf, sem):
                pltpu.make_async_copy(
                    v_hbm.at[gidx_v.at[j]], buf, sem).wait()

            def drain_scatter(j, sbuf, sems, semd):
                pltpu.make_async_copy(
                    sbuf, acc_sh.at[sidx_v.at[j]], sems).wait()
                pltpu.make_async_copy(
                    ex_v.at[j], den_sh.at[sidx_v.at[j]], semd).wait()

            def scale_scatter(j, buf, sbuf, sems, semd):
                jvec = jnp.full((16,), 0, jnp.int32) + j

                def do_edge(row):
                    exb = plsc.load_gather(
                        ex_v, [jvec, jnp.full((16,), 0, jnp.int32) + row])
                    for k in range(8):
                        sbuf[row, pl.ds(k * 16, 16)] = (
                            buf[row, pl.ds(k * 16, 16)] * exb)

                def grp_body(g, carry3):
                    for e in range(16):
                        do_edge(g * 16 + e)
                    return carry3

                lax.fori_loop(0, CC // 16, grp_body, None)
                for e in range(CC - 16 * (CC // 16)):
                    do_edge(16 * (CC // 16) + e)
                pltpu.async_copy(sbuf, acc_sh.at[sidx_v.at[j]], sems,
                                 add=True)
                pltpu.async_copy(ex_v.at[j], den_sh.at[sidx_v.at[j]],
                                 semd, add=True)

            vbufs = [(vrow0, sem0), (vrow1, sem1)]
            sbufs = [(srow0, sems0, semd0), (srow1, sems1, semd1)]
            fire(0, vrow0, sem0)
            fire(1, vrow1, sem1)
            for j in range(SCN):
                vb, vsem = vbufs[j % 2]
                sb, ssem, dsem = sbufs[j % 2]
                drain(j, vb, vsem)
                if j >= 2:
                    drain_scatter(j - 2, sb, ssem, dsem)
                scale_scatter(j, vb, sb, ssem, dsem)
                if j + 2 < SCN:
                    fire(j + 2, vb, vsem)
            sb, ssem, dsem = sbufs[(SCN - 2) % 2]
            drain_scatter(SCN - 2, sb, ssem, dsem)
            sb, ssem, dsem = sbufs[(SCN - 1) % 2]
            drain_scatter(SCN - 1, sb, ssem, dsem)

        stage(0, gidx0, sidx0, exv0, stg0)

        def super_pair(i, carry):
            s0 = 2 * i
            stage(s0 + 1, gidx1, sidx1, exv1, stg1)
            process(s0, gidx0, sidx0, exv0, stg0)

            @pl.when(s0 + 2 < NSC)
            def _():
                stage(s0 + 2, gidx0, sidx0, exv0, stg0)

            process(s0 + 1, gidx1, sidx1, exv1, stg1)
            return carry

        lax.fori_loop(0, NSC // 2, super_pair, None)
        plsc.subcore_barrier()
        for k in range(8):
            b = sid + NS * k

            @pl.when(b < NZB)
            def _():
                pltpu.sync_copy(acc_sh.at[pl.ds(b * CHUNK, CHUNK)],
                                outv_hbm.at[out_slot, pl.ds(b * CHUNK, CHUNK)])

        pltpu.sync_copy(
            den_sh.at[pl.ds(sid * DZB, DZB)],
            outd_hbm.at[pl.ds(out_slot * NDP + sid * DZB, DZB)])

    @pl.when(cid == 0)
    def _():
        run_side(vr_hbm, segrc_hbm, seglc_hbm, 0)

    @pl.when(cid == 1)
    def _():
        run_side(vl_hbm, seglc_hbm, segrc_hbm, 1)


def _edge_scatter(vr, vl, seglc, segrc, exc):
    mesh = plsc.VectorSubcoreMesh(core_axis_name="c", subcore_axis_name="s")
    f = functools.partial(
        pl.kernel,
        mesh=mesh,
        compiler_params=pltpu.CompilerParams(needs_layout_passes=False),
        out_type=[
            jax.ShapeDtypeStruct((2, N_NODE, D), jnp.float32),
            jax.ShapeDtypeStruct((2 * NDP,), jnp.float32),
        ],
        scratch_types=[
            pltpu.VMEM((SCN, CC), jnp.int32),
            pltpu.VMEM((SCN, CC), jnp.int32),
            pltpu.VMEM((SCN, CC), jnp.float32),
            pltpu.VMEM((SCN, CC), jnp.int32),
            pltpu.VMEM((SCN, CC), jnp.int32),
            pltpu.VMEM((SCN, CC), jnp.float32),
            pltpu.VMEM((CC, D), jnp.float32),
            pltpu.VMEM((CC, D), jnp.float32),
            pltpu.VMEM((CC, D), jnp.float32),
            pltpu.VMEM((CC, D), jnp.float32),
            pltpu.VMEM((DZB,), jnp.float32),
            pltpu.VMEM_SHARED((N_NODE, D), jnp.float32),
            pltpu.VMEM_SHARED((NDP,), jnp.float32),
            pltpu.SemaphoreType.DMA,
            pltpu.SemaphoreType.DMA,
            pltpu.SemaphoreType.DMA,
            pltpu.SemaphoreType.DMA,
            pltpu.SemaphoreType.DMA,
            pltpu.SemaphoreType.DMA,
            pltpu.SemaphoreType.DMA,
            pltpu.SemaphoreType.DMA,
        ],
    )(_scatter_body)
    return f(vr, vl, seglc, segrc, exc)


def _out_body(acc_ref, den_ref, wo_ref, bo_ref, o_ref):
    av = acc_ref[0]
    d = den_ref[0]
    m = lax.dot_general(av, wo_ref[...], (((1,), (1,)), ((), ())),
                        preferred_element_type=jnp.float32)
    safe = jnp.where(d > 0, d, 1.0)
    r = m / safe + bo_ref[...]
    o_ref[0] = jnp.where(r >= 0, r, 0.01 * r)


def _finalize(acc, den, wo, bo2):
    nb = 10
    rb = N_NODE // nb
    return pl.pallas_call(
        _out_body,
        grid=(2, nb),
        in_specs=[
            pl.BlockSpec((1, rb, D), lambda s, i: (s, i, 0)),
            pl.BlockSpec((1, rb, 1), lambda s, i: (s, i, 0)),
            pl.BlockSpec((D, D), lambda s, i: (0, 0)),
            pl.BlockSpec((1, D), lambda s, i: (0, 0)),
        ],
        out_specs=pl.BlockSpec((1, rb, D), lambda s, i: (s, i, 0)),
        out_shape=jax.ShapeDtypeStruct((2, N_NODE, D), jnp.float32),
    )(acc, den, wo, bo2)


def kernel(node_left, segmentation_index_left, index_left, node_right,
           segmentation_index_right, index_right, Wk, Wv, Wo, bo):
    seg_l = segmentation_index_left
    seg_r = segmentation_index_right

    a, vl, b, vr = _project(node_left, node_right, Wk, Wv)

    ex = _edge_logits(a, b, seg_l, seg_r)

    segl_c = seg_l.reshape(NS, NSC, SCN, CC)
    segr_c = seg_r.reshape(NS, NSC, SCN, CC)
    exc = ex.reshape(NS, NSC, SCN, CC)
    accv, den = _edge_scatter(vr, vl, segl_c, segr_c, exc)
    den = den.reshape(2, NDP)[:, :N_NODE]

    out = _finalize(accv, den.reshape(2, N_NODE, 1), Wo, bo.reshape(1, D))
    return (out[0], out[1])


# speedup vs baseline: 1.0575x; 1.0236x over previous
import functools

import jax
import jax.numpy as jnp
from jax import lax
from jax.experimental import pallas as pl
from jax.experimental.pallas import tpu as pltpu
from jax.experimental.pallas import tpu_sc as plsc

N_NODE = 10000
D = 128
E = 320000
INV_T = float(1.0 / (128.0 ** 0.5))

NC = 2
NS = 16
NW = NC * NS
CHUNK = 80
EB = E // NW
CB = 128
NBF = EB // CB
TB = EB - NBF * CB
EC = E // NS
NB_C = EC // CHUNK


def _proj_body(xl_ref, xr_ref, wk_ref, wv_ref, a_ref, vl_ref, b_ref, vr_ref):
    xl = xl_ref[...]
    xr = xr_ref[...]
    wk = wk_ref[...]
    wv = wv_ref[...]
    dn = (((1,), (1,)), ((), ()))
    a_ref[...] = lax.dot_general(xl, wk, dn,
                                 preferred_element_type=jnp.float32)
    vl_ref[...] = lax.dot_general(xl, wv, dn,
                                  preferred_element_type=jnp.float32)
    b_ref[...] = lax.dot_general(xr, wk, dn,
                                 preferred_element_type=jnp.float32)
    vr_ref[...] = lax.dot_general(xr, wv, dn,
                                  preferred_element_type=jnp.float32)


def _project(xl, xr, wk, wv):
    nb = 10
    rb = N_NODE // nb
    blk = pl.BlockSpec((rb, D), lambda i: (i, 0))
    wblk = pl.BlockSpec((D, D), lambda i: (0, 0))
    osh = jax.ShapeDtypeStruct((N_NODE, D), jnp.float32)
    return pl.pallas_call(
        _proj_body,
        grid=(nb,),
        in_specs=[blk, blk, wblk, wblk],
        out_specs=[blk, blk, blk, blk],
        out_shape=[osh, osh, osh, osh],
    )(xl, xr, wk, wv)


def _logits_body(a_hbm, b_hbm, segl_hbm, segr_hbm, ex_hbm,
                 idxl_v, idxr_v, ex_v, arow0, brow0, arow1, brow1, sbuf_v,
                 sem0, sem1):
    cid = lax.axis_index("c")
    sid = lax.axis_index("s")
    wid = sid * NC + cid
    pltpu.sync_copy(segl_hbm.at[pl.ds(wid * EB, EB)], idxl_v)
    pltpu.sync_copy(segr_hbm.at[pl.ds(wid * EB, EB)], idxr_v)
    lanes = lax.iota(jnp.int32, 16)

    def fire(c, ar, br, sem):
        pltpu.async_copy(
            a_hbm.at[idxl_v.at[pl.ds(c * CB, CB)]], ar, sem)
        pltpu.async_copy(
            b_hbm.at[idxr_v.at[pl.ds(c * CB, CB)]], br, sem)

    def drain(c, ar, br, sem):
        pltpu.make_async_copy(
            a_hbm.at[idxl_v.at[pl.ds(c * CB, CB)]], ar, sem).wait()
        pltpu.make_async_copy(
            b_hbm.at[idxr_v.at[pl.ds(c * CB, CB)]], br, sem).wait()

    def compute(base, ar, br, ngrp):
        def grp_body(g, carry2):
            r0 = g * 16
            for e in range(16):
                row = r0 + e
                p = ar[row, pl.ds(0, 16)] * br[row, pl.ds(0, 16)]
                for k in range(1, 8):
                    p = p + (ar[row, pl.ds(k * 16, 16)]
                             * br[row, pl.ds(k * 16, 16)])
                sbuf_v[e, pl.ds(0, 16)] = p
            tv = plsc.load_gather(
                sbuf_v, [lanes, jnp.full((16,), 0, jnp.int32)])
            for l in range(1, 16):
                tv = tv + plsc.load_gather(
                    sbuf_v, [lanes, jnp.full((16,), l, jnp.int32)])
            ex_v[pl.ds(base + r0, 16)] = jnp.exp(tv * INV_T)
            return carry2

        lax.fori_loop(0, ngrp, grp_body, None)

    fire(0, arow0, brow0, sem0)

    def pair_body(i, carry):
        c0 = 2 * i
        fire(c0 + 1, arow1, brow1, sem1)
        drain(c0, arow0, brow0, sem0)
        compute(c0 * CB, arow0, brow0, CB // 16)

        @pl.when(c0 + 2 < NBF)
        def _():
            fire(c0 + 2, arow0, brow0, sem0)

        drain(c0 + 1, arow1, brow1, sem1)
        compute((c0 + 1) * CB, arow1, brow1, CB // 16)
        return carry

    lax.fori_loop(0, NBF // 2, pair_body, None)
    ar_t = arow0.at[pl.ds(0, TB)]
    br_t = brow0.at[pl.ds(0, TB)]
    pltpu.async_copy(
        a_hbm.at[idxl_v.at[pl.ds(NBF * CB, TB)]], ar_t, sem0)
    pltpu.async_copy(
        b_hbm.at[idxr_v.at[pl.ds(NBF * CB, TB)]], br_t, sem0)
    pltpu.make_async_copy(
        a_hbm.at[idxl_v.at[pl.ds(NBF * CB, TB)]], ar_t, sem0).wait()
    pltpu.make_async_copy(
        b_hbm.at[idxr_v.at[pl.ds(NBF * CB, TB)]], br_t, sem0).wait()
    compute(NBF * CB, arow0, brow0, TB // 16)
    pltpu.sync_copy(ex_v, ex_hbm.at[pl.ds(wid * EB, EB)])


def _edge_logits(a, b, segl3, segr3):
    mesh = plsc.VectorSubcoreMesh(core_axis_name="c", subcore_axis_name="s")
    f = functools.partial(
        pl.kernel,
        mesh=mesh,
        compiler_params=pltpu.CompilerParams(needs_layout_passes=False),
        out_type=jax.ShapeDtypeStruct((E,), jnp.float32),
        scratch_types=[
            pltpu.VMEM((EB,), jnp.int32),
            pltpu.VMEM((EB,), jnp.int32),
            pltpu.VMEM((EB,), jnp.float32),
            pltpu.VMEM((CB, D), jnp.float32),
            pltpu.VMEM((CB, D), jnp.float32),
            pltpu.VMEM((CB, D), jnp.float32),
            pltpu.VMEM((CB, D), jnp.float32),
            pltpu.VMEM((16, 16), jnp.float32),
            pltpu.SemaphoreType.DMA,
            pltpu.SemaphoreType.DMA,
        ],
    )(_logits_body)
    return f(a, b, segl3, segr3)


CC = 80
NBC = EC // CC
NZB = N_NODE // CC
SCN = 10
NSC = NBC // SCN
DZB = 640
NDP = NS * DZB


def _scatter_body(vr_hbm, vl_hbm, seglc_hbm, segrc_hbm, exc_hbm,
                  outv_hbm, outd_hbm,
                  gidx_v, sidx_v, ex_v,
                  vrow0, vrow1, srow0, srow1, zden_v,
                  acc_sh, den_sh,
                  sem0, sem1, sems0, sems1, semd0, semd1):
    cid = lax.axis_index("c")
    sid = lax.axis_index("s")

    def zrow_body(i, carry):
        for k in range(D // 16):
            vrow0[i, pl.ds(k * 16, 16)] = jnp.zeros((16,), jnp.float32)
        return carry

    lax.fori_loop(0, CC, zrow_body, None)

    def zden_body(i, carry):
        zden_v[pl.ds(i * 16, 16)] = jnp.zeros((16,), jnp.float32)
        return carry

    lax.fori_loop(0, DZB // 16, zden_body, None)

    for k in range(8):
        b = sid + NS * k

        @pl.when(b < NZB)
        def _():
            pltpu.sync_copy(vrow0, acc_sh.at[pl.ds(b * CC, CC)])

    pltpu.sync_copy(zden_v, den_sh.at[pl.ds(sid * DZB, DZB)])

    plsc.subcore_barrier()

    def run_side(v_hbm, g4d, s4d, out_slot):
        def process(s):
            pltpu.sync_copy(g4d.at[sid, s], gidx_v)
            pltpu.sync_copy(s4d.at[sid, s], sidx_v)
            pltpu.sync_copy(exc_hbm.at[sid, s], ex_v)

            def fire(j, buf, sem):
                pltpu.async_copy(v_hbm.at[gidx_v.at[j]], buf, sem)

            def drain(j, buf, sem):
                pltpu.make_async_copy(
                    v_hbm.at[gidx_v.at[j]], buf, sem).wait()

            def drain_scatter(j, sbuf, sems, semd):
                pltpu.make_async_copy(
                    sbuf, acc_sh.at[sidx_v.at[j]], sems).wait()
                pltpu.make_async_copy(
                    ex_v.at[j], den_sh.at[sidx_v.at[j]], semd).wait()

            def scale_scatter(j, buf, sbuf, sems, semd):
                jvec = jnp.full((16,), 0, jnp.int32) + j

                def do_edge(row):
                    exb = plsc.load_gather(
                        ex_v, [jvec, jnp.full((16,), 0, jnp.int32) + row])
                    for k in range(8):
                        sbuf[row, pl.ds(k * 16, 16)] = (
                            buf[row, pl.ds(k * 16, 16)] * exb)

                def grp_body(g, carry3):
                    for e in range(16):
                        do_edge(g * 16 + e)
                    return carry3

                lax.fori_loop(0, CC // 16, grp_body, None)
                for e in range(CC - 16 * (CC // 16)):
                    do_edge(16 * (CC // 16) + e)
                pltpu.async_copy(sbuf, acc_sh.at[sidx_v.at[j]], sems,
                                 add=True)
                pltpu.async_copy(ex_v.at[j], den_sh.at[sidx_v.at[j]],
                                 semd, add=True)

            fire(0, vrow0, sem0)

            def pair_body(p, carry2):
                j0 = 2 * p
                fire(j0 + 1, vrow1, sem1)
                drain(j0, vrow0, sem0)

                @pl.when(j0 >= 2)
                def _():
                    drain_scatter(j0 - 2, srow0, sems0, semd0)

                scale_scatter(j0, vrow0, srow0, sems0, semd0)

                @pl.when(j0 + 2 < SCN)
                def _():
                    fire(j0 + 2, vrow0, sem0)

                drain(j0 + 1, vrow1, sem1)

                @pl.when(j0 >= 2)
                def _():
                    drain_scatter(j0 - 1, srow1, sems1, semd1)

                scale_scatter(j0 + 1, vrow1, srow1, sems1, semd1)
                return carry2

            lax.fori_loop(0, SCN // 2, pair_body, None)
            drain_scatter(SCN - 2, srow0, sems0, semd0)
            drain_scatter(SCN - 1, srow1, sems1, semd1)

        def sc_body(s, carry):
            process(s)
            return carry

        lax.fori_loop(0, NSC, sc_body, None)
        plsc.subcore_barrier()
        for k in range(8):
            b = sid + NS * k

            @pl.when(b < NZB)
            def _():
                pltpu.sync_copy(acc_sh.at[pl.ds(b * CHUNK, CHUNK)],
                                outv_hbm.at[out_slot, pl.ds(b * CHUNK, CHUNK)])

        pltpu.sync_copy(
            den_sh.at[pl.ds(sid * DZB, DZB)],
            outd_hbm.at[pl.ds(out_slot * NDP + sid * DZB, DZB)])

    @pl.when(cid == 0)
    def _():
        run_side(vr_hbm, segrc_hbm, seglc_hbm, 0)

    @pl.when(cid == 1)
    def _():
        run_side(vl_hbm, seglc_hbm, segrc_hbm, 1)


def _edge_scatter(vr, vl, seglc, segrc, exc):
    mesh = plsc.VectorSubcoreMesh(core_axis_name="c", subcore_axis_name="s")
    f = functools.partial(
        pl.kernel,
        mesh=mesh,
        compiler_params=pltpu.CompilerParams(needs_layout_passes=False),
        out_type=[
            jax.ShapeDtypeStruct((2, N_NODE, D), jnp.float32),
            jax.ShapeDtypeStruct((2 * NDP,), jnp.float32),
        ],
        scratch_types=[
            pltpu.VMEM((SCN, CC), jnp.int32),
            pltpu.VMEM((SCN, CC), jnp.int32),
            pltpu.VMEM((SCN, CC), jnp.float32),
            pltpu.VMEM((CC, D), jnp.float32),
            pltpu.VMEM((CC, D), jnp.float32),
            pltpu.VMEM((CC, D), jnp.float32),
            pltpu.VMEM((CC, D), jnp.float32),
            pltpu.VMEM((DZB,), jnp.float32),
            pltpu.VMEM_SHARED((N_NODE, D), jnp.float32),
            pltpu.VMEM_SHARED((NDP,), jnp.float32),
            pltpu.SemaphoreType.DMA,
            pltpu.SemaphoreType.DMA,
            pltpu.SemaphoreType.DMA,
            pltpu.SemaphoreType.DMA,
            pltpu.SemaphoreType.DMA,
            pltpu.SemaphoreType.DMA,
        ],
    )(_scatter_body)
    return f(vr, vl, seglc, segrc, exc)


def _out_body(acc_ref, den_ref, wo_ref, bo_ref, o_ref):
    av = acc_ref[0]
    d = den_ref[0]
    m = lax.dot_general(av, wo_ref[...], (((1,), (1,)), ((), ())),
                        preferred_element_type=jnp.float32)
    safe = jnp.where(d > 0, d, 1.0)
    r = m / safe + bo_ref[...]
    o_ref[0] = jnp.where(r >= 0, r, 0.01 * r)


def _finalize(acc, den, wo, bo2):
    nb = 10
    rb = N_NODE // nb
    return pl.pallas_call(
        _out_body,
        grid=(2, nb),
        in_specs=[
            pl.BlockSpec((1, rb, D), lambda s, i: (s, i, 0)),
            pl.BlockSpec((1, rb, 1), lambda s, i: (s, i, 0)),
            pl.BlockSpec((D, D), lambda s, i: (0, 0)),
            pl.BlockSpec((1, D), lambda s, i: (0, 0)),
        ],
        out_specs=pl.BlockSpec((1, rb, D), lambda s, i: (s, i, 0)),
        out_shape=jax.ShapeDtypeStruct((2, N_NODE, D), jnp.float32),
    )(acc, den, wo, bo2)


def kernel(node_left, segmentation_index_left, index_left, node_right,
           segmentation_index_right, index_right, Wk, Wv, Wo, bo):
    seg_l = segmentation_index_left
    seg_r = segmentation_index_right

    a, vl, b, vr = _project(node_left, node_right, Wk, Wv)

    ex = _edge_logits(a, b, seg_l, seg_r)

    segl_c = seg_l.reshape(NS, NSC, SCN, CC)
    segr_c = seg_r.reshape(NS, NSC, SCN, CC)
    exc = ex.reshape(NS, NSC, SCN, CC)
    accv, den = _edge_scatter(vr, vl, segl_c, segr_c, exc)
    den = den.reshape(2, NDP)[:, :N_NODE]

    out = _finalize(accv, den.reshape(2, N_NODE, 1), Wo, bo.reshape(1, D))
    return (out[0], out[1])
